# baseline (speedup 1.0000x reference)
import jax
import jax.numpy as jnp
from jax.experimental import pallas as pl

_N_GAME = 50000
_N_STATE = 50000
_NREL = 3
_K = 3
_HID = 64


def _seg(d, idx, n):
    return jax.ops.segment_sum(d, idx, num_segments=n)


def _final_linear_pallas(sx, W, b):
    n, hid = sx.shape
    out_dim = W.shape[1]
    blk = 1000

    def body(x_ref, w_ref, b_ref, o_ref):
        o_ref[...] = x_ref[...] @ w_ref[...] + b_ref[...]

    return pl.pallas_call(
        body,
        grid=(n // blk,),
        in_specs=[
            pl.BlockSpec((blk, hid), lambda i: (i, 0)),
            pl.BlockSpec((hid, out_dim), lambda i: (0, 0)),
            pl.BlockSpec((1, out_dim), lambda i: (0, 0)),
        ],
        out_specs=pl.BlockSpec((blk, out_dim), lambda i: (i, 0)),
        out_shape=jax.ShapeDtypeStruct((n, out_dim), jnp.float32),
    )(sx, W, b.reshape(1, -1))


def kernel(game_x, state_x, edge_index_v_v, edge_type_v_v,
           edge_index_history_v_s, edge_attr_history_v_s,
           edge_index_in_v_s, edge_index_s_s, params):
    p = params
    f32 = jnp.float32
    src_vv, dst_vv = edge_index_v_v[0], edge_index_v_v[1]
    src_hs, dst_hs = edge_index_history_v_s[0], edge_index_history_v_s[1]
    src_in, dst_in = edge_index_in_v_s[0], edge_index_in_v_s[1]
    src_ss, dst_ss = edge_index_s_s[0], edge_index_s_s[1]
    x = game_x

    et = edge_type_v_v
    cnt_r = _seg(jax.nn.one_hot(et, _NREL, dtype=f32), dst_vv, _N_GAME)
    deg_vv = jnp.sum(cnt_r, axis=1)
    cnt_hs = _seg(jnp.ones_like(src_hs, f32), dst_hs, _N_STATE)
    cnt_in = _seg(jnp.ones_like(src_in, f32), dst_in, _N_STATE)
    cnt_ss = _seg(jnp.ones_like(src_ss, f32), dst_ss, _N_STATE)

    dis_vv = jnp.where(deg_vv > 0, deg_vv ** -0.5, 0.0)
    xs = x[src_vv]
    onehot = jax.nn.one_hot(et, _NREL, dtype=f32)
    S = _seg(xs[:, None, :] * onehot[:, :, None], dst_vv, _N_GAME)
    rgcn = x @ p['rgcn_root'] + p['rgcn_b']
    for r in range(_NREL):
        rgcn = rgcn + (S[:, r] / jnp.clip(cnt_r[:, r], 1.0)[:, None]) @ p['rgcn_W'][r]

    tag = x @ p['tag1_W'][0]
    h = x
    for k in range(1, _K + 1):
        hp = dis_vv[:, None] * h
        h = dis_vv[:, None] * _seg(hp[src_vv], dst_vv, _N_GAME)
        tag = tag + h @ p['tag1_W'][k]
    gx = rgcn + tag + p['tag1_b']

    gxs = gx[src_hs]
    agg_w = _seg(edge_attr_history_v_s[:, None] * gxs, dst_hs, _N_STATE)
    agg_u = _seg(gxs, dst_hs, _N_STATE)
    sx = jax.nn.relu(agg_w @ p['gc3_Wrel'] + p['gc3_brel'] + state_x @ p['gc3_Wroot'])
    mean_hs = agg_u / jnp.clip(cnt_hs, 1.0)[:, None]
    sx = jax.nn.relu(mean_hs @ p['s32_Wl'] + p['s32_bl'] + sx @ p['s32_Wr'])

    mean_in = _seg(gx[src_in], dst_in, _N_STATE) / jnp.clip(cnt_in, 1.0)[:, None]
    sx = jax.nn.relu(mean_in @ p['s4_Wl'] + p['s4_bl'] + sx @ p['s4_Wr'])
    sx = jax.nn.relu(mean_in @ p['s42_Wl'] + p['s42_bl'] + sx @ p['s42_Wr'])

    dis_ss = jnp.where(cnt_ss > 0, cnt_ss ** -0.5, 0.0)
    tag2 = sx @ p['tag2_W'][0]
    h = sx
    for k in range(1, _K + 1):
        hp = dis_ss[:, None] * h
        h = dis_ss[:, None] * _seg(hp[src_ss], dst_ss, _N_STATE)
        tag2 = tag2 + h @ p['tag2_W'][k]
    sx = jax.nn.relu(tag2 + p['tag2_b'])

    mean5 = _seg(sx[src_ss], dst_ss, _N_STATE) / jnp.clip(cnt_ss, 1.0)[:, None]
    sx = jax.nn.relu(mean5 @ p['s5_Wl'] + p['s5_bl'] + sx @ p['s5_Wr'])

    return _final_linear_pallas(sx, p['lin_W'], p['lin_b'])


# speedup vs baseline: 14.2445x; 14.2445x over previous
import functools

import jax
import jax.numpy as jnp
from jax import lax
from jax.experimental import pallas as pl
from jax.experimental.pallas import tpu as pltpu
from jax.experimental.pallas import tpu_sc as plsc

_N = 50000
_NREL = 3
_K = 3
_NACC = 50176
_TRASH = _NACC - _N

_MESH = plsc.VectorSubcoreMesh(core_axis_name="c", subcore_axis_name="s")
_PARAMS = pltpu.CompilerParams(use_tc_tiling_on_sc=False)

_EP = 819200
_ROWS = _EP // 128


def _zeros16():
    return jnp.zeros((16,), jnp.float32)


def _segf_body(t4_ref, src_ref, dst_ref, ea_ref, flag_ref, out_ref,
               src_v, dst_v, rows_v, w16_v, fl_s, zb_v, acc_sh, sem):
    c = lax.axis_index("c")
    s = lax.axis_index("s")
    for i in range(64):
        zb_v[i, pl.ds(0, 16)] = _zeros16()
    pltpu.sync_copy(flag_ref, fl_s)
    wflag = fl_s[pl.ds(0, 16)][0] != 0
    tb = s * (_NACC // 16)
    row0 = s * (_ROWS // 16)

    for rnd in range(2):
        def zl(j, _):
            pltpu.sync_copy(zb_v, acc_sh.at[pl.ds(tb + j * 64, 64)])
            return 0
        lax.fori_loop(0, _NACC // 16 // 64, zl, 0)
        plsc.subcore_barrier()
        off = (c + 2 * rnd) * _N

        def win(t, _):
            r = row0 + t * 16
            pltpu.sync_copy(src_ref.at[pl.ds(r, 16)], src_v)
            pltpu.sync_copy(dst_ref.at[rnd, c, pl.ds(r, 16)], dst_v)
            for i in range(16):
                for g in range(8):
                    src_v[i, pl.ds(g * 16, 16)] = (
                        src_v[i, pl.ds(g * 16, 16)] + off)
            hs = [pltpu.async_copy(t4_ref.at[src_v.at[j]],
                                   rows_v.at[pl.ds(j * 128, 128)], sem)
                  for j in range(16)]
            for h in hs:
                h.wait()

            @pl.when(wflag)
            def _weight():
                pltpu.sync_copy(ea_ref.at[pl.ds(r * 128, 2048)], w16_v)

                def ml(e, _):
                    rows_v[e, :] = rows_v[e, :] * w16_v[e, :]
                    return 0
                lax.fori_loop(0, 2048, ml, 0, unroll=8)

            for j in range(16):
                pltpu.sync_copy(rows_v.at[pl.ds(j * 128, 128)],
                                acc_sh.at[dst_v.at[j]], add=True)
            return 0
        lax.fori_loop(0, _ROWS // 16 // 16, win, 0)
        plsc.subcore_barrier()
        pltpu.sync_copy(acc_sh.at[pl.ds(tb, _NACC // 16)],
                        out_ref.at[rnd, c, pl.ds(tb, _NACC // 16)])
        plsc.subcore_barrier()


_segf = functools.partial(
    pl.kernel,
    out_type=jax.ShapeDtypeStruct((2, 2, _NACC, 16), jnp.float32),
    mesh=_MESH, compiler_params=_PARAMS,
    scratch_types=[
        pltpu.VMEM((16, 128), jnp.int32),
        pltpu.VMEM((16, 128), jnp.int32),
        pltpu.VMEM((2048, 16), jnp.float32),
        pltpu.VMEM((2048, 16), jnp.float32),
        pltpu.VMEM((16,), jnp.int32),
        pltpu.VMEM((64, 16), jnp.float32),
        pltpu.VMEM_SHARED((_NACC, 16), jnp.float32),
        pltpu.SemaphoreType.DMA,
    ],
)(_segf_body)

def _segq_call(t4, src_p, dst4, ea):
    return _segf(t4, src_p, dst4, ea, jnp.zeros((16,), jnp.int32))


def _segqw_call(t4, src_p, dst4, ea16):
    return _segf(t4, src_p, dst4, ea16, jnp.ones((16,), jnp.int32))


def _final_linear_pallas(sx, W, b):
    n, hid = sx.shape
    out_dim = W.shape[1]
    blk = 1000

    def body(x_ref, w_ref, b_ref, o_ref):
        o_ref[...] = x_ref[...] @ w_ref[...] + b_ref[...]

    return pl.pallas_call(
        body,
        grid=(n // blk,),
        in_specs=[
            pl.BlockSpec((blk, hid), lambda i: (i, 0)),
            pl.BlockSpec((hid, out_dim), lambda i: (0, 0)),
            pl.BlockSpec((1, out_dim), lambda i: (0, 0)),
        ],
        out_specs=pl.BlockSpec((blk, out_dim), lambda i: (i, 0)),
        out_shape=jax.ShapeDtypeStruct((n, out_dim), jnp.float32),
    )(sx, W, b.reshape(1, -1))


def _pad_src(src):
    pad = _EP - src.shape[0]
    ps = jnp.arange(pad, dtype=jnp.int32) % _N
    return jnp.concatenate([src.astype(jnp.int32), ps]).reshape(_ROWS, 128)


def _pad_dst(dst):
    pad = _EP - dst.shape[0]
    pd = _N + jnp.arange(pad, dtype=jnp.int32) % _TRASH
    return jnp.concatenate([dst.astype(jnp.int32), pd]).reshape(_ROWS, 128)


def _dst4(d00, d01, d10, d11):
    return jnp.stack([jnp.stack([d00, d01]), jnp.stack([d10, d11])])


def _quad_table(x):
    return jnp.concatenate([x[:, 0:16], x[:, 16:32], x[:, 32:48],
                            x[:, 48:64]], axis=0)


def _quad_out(o):
    return jnp.concatenate([o[0, 0, :_N], o[0, 1, :_N],
                            o[1, 0, :_N], o[1, 1, :_N]], axis=1)


def _seg64(x, src_p, dst4, ea):
    return _quad_out(_segq_call(_quad_table(x), src_p, dst4, ea))


def kernel(game_x, state_x, edge_index_v_v, edge_type_v_v,
           edge_index_history_v_s, edge_attr_history_v_s,
           edge_index_in_v_s, edge_index_s_s, params):
    p = params
    f32 = jnp.float32
    src_vv, dst_vv = edge_index_v_v[0], edge_index_v_v[1]
    src_hs, dst_hs = edge_index_history_v_s[0], edge_index_history_v_s[1]
    src_in, dst_in = edge_index_in_v_s[0], edge_index_in_v_s[1]
    src_ss, dst_ss = edge_index_s_s[0], edge_index_s_s[1]
    et = edge_type_v_v.astype(jnp.int32)

    srcg = _pad_src(src_vv)
    srch = _pad_src(src_hs)
    srci = _pad_src(src_in)
    srcs = _pad_src(src_ss)
    dhop = _pad_dst(dst_vv)
    dhs = _pad_dst(dst_hs)
    din = _pad_dst(dst_in)
    dss = _pad_dst(dst_ss)
    dhs4 = _dst4(dhs, dhs, dhs, dhs)
    din4 = _dst4(din, din, din, din)
    dss4 = _dst4(dss, dss, dss, dss)
    padf = _EP - src_hs.shape[0]
    ea16 = jnp.broadcast_to(
        jnp.concatenate([edge_attr_history_v_s.astype(f32),
                         jnp.zeros((padf,), f32)])[:, None],
        (_EP, 16))

    ones4 = jnp.ones((4 * _N, 16), f32)
    cnto = _segq_call(ones4, srch, _dst4(dhs, din, dss, dhop), ea16)
    cnt_hs = cnto[0, 0, :_N, 0]
    cnt_in = cnto[0, 1, :_N, 0]
    cnt_ss = cnto[1, 0, :_N, 0]
    deg_vv = cnto[1, 1, :_N, 0]

    dis_vv = jnp.where(deg_vv > 0, deg_vv ** -0.5, 0.0)
    x16 = jnp.concatenate([game_x, jnp.ones((_N, 1), f32),
                           jnp.zeros((_N, 10), f32)], axis=1)
    xp16 = dis_vv[:, None] * x16
    tr_e = _N + (jnp.arange(src_vv.shape[0], dtype=jnp.int32) % _TRASH)
    drel = [_pad_dst(jnp.where(et == r, dst_vv.astype(jnp.int32), tr_e))
            for r in range(_NREL)]
    gameo = _segq_call(jnp.concatenate([x16, x16, x16, xp16], axis=0),
                       srcg, _dst4(drel[0], drel[1], drel[2], dhop), ea16)
    rgcn = game_x @ p['rgcn_root'] + p['rgcn_b']
    quads = [gameo[0, 0], gameo[0, 1], gameo[1, 0], gameo[1, 1]]
    for r in range(_NREL):
        S_r = quads[r][:_N, :5]
        cnt_r = quads[r][:_N, 5]
        rgcn = rgcn + (S_r / jnp.clip(cnt_r, 1.0)[:, None]) @ p['rgcn_W'][r]

    zeros3n = jnp.zeros((3 * _N, 16), f32)
    tr_full = _pad_dst(tr_e)
    dhop4 = _dst4(dhop, tr_full, tr_full, tr_full)
    tag = game_x @ p['tag1_W'][0]
    h = dis_vv[:, None] * quads[3][:_N, :5]
    tag = tag + h @ p['tag1_W'][1]
    for k in range(2, _K + 1):
        hp16 = jnp.concatenate([dis_vv[:, None] * h,
                                jnp.zeros((_N, 11), f32)], axis=1)
        hsum = _segq_call(jnp.concatenate([hp16, zeros3n], axis=0),
                          srcg, dhop4, ea16)
        h = dis_vv[:, None] * hsum[0, 0, :_N, :5]
        tag = tag + h @ p['tag1_W'][k]
    gx = rgcn + tag + p['tag1_b']

    t4_gx = _quad_table(gx)
    agg_w = _quad_out(_segqw_call(t4_gx, srch, dhs4, ea16))
    agg_u = _quad_out(_segq_call(t4_gx, srch, dhs4, ea16))

    sx = jax.nn.relu(agg_w @ p['gc3_Wrel'] + p['gc3_brel']
                     + state_x @ p['gc3_Wroot'])
    mean_hs = agg_u / jnp.clip(cnt_hs, 1.0)[:, None]
    sx = jax.nn.relu(mean_hs @ p['s32_Wl'] + p['s32_bl'] + sx @ p['s32_Wr'])

    mean_in = _seg64(gx, srci, din4, ea16) / jnp.clip(cnt_in, 1.0)[:, None]
    sx = jax.nn.relu(mean_in @ p['s4_Wl'] + p['s4_bl'] + sx @ p['s4_Wr'])
    sx = jax.nn.relu(mean_in @ p['s42_Wl'] + p['s42_bl'] + sx @ p['s42_Wr'])

    dis_ss = jnp.where(cnt_ss > 0, cnt_ss ** -0.5, 0.0)
    tag2 = sx @ p['tag2_W'][0]
    h = sx
    for k in range(1, _K + 1):
        hp = dis_ss[:, None] * h
        h = dis_ss[:, None] * _seg64(hp, srcs, dss4, ea16)
        tag2 = tag2 + h @ p['tag2_W'][k]
    sx = jax.nn.relu(tag2 + p['tag2_b'])

    mean5 = _seg64(sx, srcs, dss4, ea16) / jnp.clip(cnt_ss, 1.0)[:, None]
    sx = jax.nn.relu(mean5 @ p['s5_Wl'] + p['s5_bl'] + sx @ p['s5_Wr'])

    return _final_linear_pallas(sx, p['lin_W'], p['lin_b'])


# speedup vs baseline: 15.1944x; 1.0667x over previous
import functools

import jax
import jax.numpy as jnp
from jax import lax
from jax.experimental import pallas as pl
from jax.experimental.pallas import tpu as pltpu
from jax.experimental.pallas import tpu_sc as plsc

_N = 50000
_NREL = 3
_K = 3
_NACC = 50176
_TRASH = _NACC - _N

_MESH = plsc.VectorSubcoreMesh(core_axis_name="c", subcore_axis_name="s")
_PARAMS = pltpu.CompilerParams(use_tc_tiling_on_sc=False)

_EP = 819200
_ROWS = _EP // 128


def _zeros16():
    return jnp.zeros((16,), jnp.float32)


def _segf_body(t4_ref, src_ref, dst_ref, ea_ref, flag_ref, out_ref,
               src_v, dst_v, rows_v, w16_v, fl_s, zb_v, acc_sh, sem, sem2):
    c = lax.axis_index("c")
    s = lax.axis_index("s")
    for i in range(64):
        zb_v[i, pl.ds(0, 16)] = _zeros16()
    pltpu.sync_copy(flag_ref, fl_s)
    wflag = fl_s[pl.ds(0, 16)][0] != 0
    tb = s * (_NACC // 16)
    row0 = s * (_ROWS // 16)

    for rnd in range(2):
        def zl(j, _):
            pltpu.sync_copy(zb_v, acc_sh.at[pl.ds(tb + j * 64, 64)])
            return 0
        lax.fori_loop(0, _NACC // 16 // 64, zl, 0)
        plsc.subcore_barrier()
        off = (c + 2 * rnd) * _N

        def win(t, _):
            r = row0 + t * 16
            pltpu.sync_copy(src_ref.at[pl.ds(r, 16)], src_v)
            pltpu.sync_copy(dst_ref.at[rnd, c, pl.ds(r, 16)], dst_v)
            for i in range(16):
                for g in range(8):
                    src_v[i, pl.ds(g * 16, 16)] = (
                        src_v[i, pl.ds(g * 16, 16)] + off)
            hs = [pltpu.async_copy(t4_ref.at[src_v.at[j]],
                                   rows_v.at[pl.ds(j * 128, 128)], sem)
                  for j in range(16)]
            for h in hs:
                h.wait()

            @pl.when(wflag)
            def _weight():
                pltpu.sync_copy(ea_ref.at[pl.ds(r * 128, 2048)], w16_v)

                def ml(e, _):
                    rows_v[e, :] = rows_v[e, :] * w16_v[e, :]
                    return 0
                lax.fori_loop(0, 2048, ml, 0, unroll=8)

            ss = [pltpu.async_copy(rows_v.at[pl.ds(j * 128, 128)],
                                   acc_sh.at[dst_v.at[j]], sem2, add=True)
                  for j in range(16)]
            for h2 in ss:
                h2.wait()
            return 0
        lax.fori_loop(0, _ROWS // 16 // 16, win, 0)
        plsc.subcore_barrier()
        pltpu.sync_copy(acc_sh.at[pl.ds(tb, _NACC // 16)],
                        out_ref.at[rnd, c, pl.ds(tb, _NACC // 16)])
        plsc.subcore_barrier()


_segf = functools.partial(
    pl.kernel,
    out_type=jax.ShapeDtypeStruct((2, 2, _NACC, 16), jnp.float32),
    mesh=_MESH, compiler_params=_PARAMS,
    scratch_types=[
        pltpu.VMEM((16, 128), jnp.int32),
        pltpu.VMEM((16, 128), jnp.int32),
        pltpu.VMEM((2048, 16), jnp.float32),
        pltpu.VMEM((2048, 16), jnp.float32),
        pltpu.VMEM((16,), jnp.int32),
        pltpu.VMEM((64, 16), jnp.float32),
        pltpu.VMEM_SHARED((_NACC, 16), jnp.float32),
        pltpu.SemaphoreType.DMA,
        pltpu.SemaphoreType.DMA,
    ],
)(_segf_body)

def _segq_call(t4, src_p, dst4, ea):
    return _segf(t4, src_p, dst4, ea, jnp.zeros((16,), jnp.int32))


def _segqw_call(t4, src_p, dst4, ea16):
    return _segf(t4, src_p, dst4, ea16, jnp.ones((16,), jnp.int32))


def _final_linear_pallas(sx, W, b):
    n, hid = sx.shape
    out_dim = W.shape[1]
    blk = 1000

    def body(x_ref, w_ref, b_ref, o_ref):
        o_ref[...] = x_ref[...] @ w_ref[...] + b_ref[...]

    return pl.pallas_call(
        body,
        grid=(n // blk,),
        in_specs=[
            pl.BlockSpec((blk, hid), lambda i: (i, 0)),
            pl.BlockSpec((hid, out_dim), lambda i: (0, 0)),
            pl.BlockSpec((1, out_dim), lambda i: (0, 0)),
        ],
        out_specs=pl.BlockSpec((blk, out_dim), lambda i: (i, 0)),
        out_shape=jax.ShapeDtypeStruct((n, out_dim), jnp.float32),
    )(sx, W, b.reshape(1, -1))


def _pad_src(src):
    pad = _EP - src.shape[0]
    ps = jnp.arange(pad, dtype=jnp.int32) % _N
    return jnp.concatenate([src.astype(jnp.int32), ps]).reshape(_ROWS, 128)


def _pad_dst(dst):
    pad = _EP - dst.shape[0]
    pd = _N + jnp.arange(pad, dtype=jnp.int32) % _TRASH
    return jnp.concatenate([dst.astype(jnp.int32), pd]).reshape(_ROWS, 128)


def _dst4(d00, d01, d10, d11):
    return jnp.stack([jnp.stack([d00, d01]), jnp.stack([d10, d11])])


def _quad_table(x):
    return jnp.concatenate([x[:, 0:16], x[:, 16:32], x[:, 32:48],
                            x[:, 48:64]], axis=0)


def _quad_out(o):
    return jnp.concatenate([o[0, 0, :_N], o[0, 1, :_N],
                            o[1, 0, :_N], o[1, 1, :_N]], axis=1)


def _seg64(x, src_p, dst4, ea):
    return _quad_out(_segq_call(_quad_table(x), src_p, dst4, ea))


def kernel(game_x, state_x, edge_index_v_v, edge_type_v_v,
           edge_index_history_v_s, edge_attr_history_v_s,
           edge_index_in_v_s, edge_index_s_s, params):
    p = params
    f32 = jnp.float32
    src_vv, dst_vv = edge_index_v_v[0], edge_index_v_v[1]
    src_hs, dst_hs = edge_index_history_v_s[0], edge_index_history_v_s[1]
    src_in, dst_in = edge_index_in_v_s[0], edge_index_in_v_s[1]
    src_ss, dst_ss = edge_index_s_s[0], edge_index_s_s[1]
    et = edge_type_v_v.astype(jnp.int32)

    srcg = _pad_src(src_vv)
    srch = _pad_src(src_hs)
    srci = _pad_src(src_in)
    srcs = _pad_src(src_ss)
    dhop = _pad_dst(dst_vv)
    dhs = _pad_dst(dst_hs)
    din = _pad_dst(dst_in)
    dss = _pad_dst(dst_ss)
    dhs4 = _dst4(dhs, dhs, dhs, dhs)
    din4 = _dst4(din, din, din, din)
    dss4 = _dst4(dss, dss, dss, dss)
    padf = _EP - src_hs.shape[0]
    ea16 = jnp.broadcast_to(
        jnp.concatenate([edge_attr_history_v_s.astype(f32),
                         jnp.zeros((padf,), f32)])[:, None],
        (_EP, 16))

    ones4 = jnp.ones((4 * _N, 16), f32)
    cnto = _segq_call(ones4, srch, _dst4(dhs, din, dss, dhop), ea16)
    cnt_hs = cnto[0, 0, :_N, 0]
    cnt_in = cnto[0, 1, :_N, 0]
    cnt_ss = cnto[1, 0, :_N, 0]
    deg_vv = cnto[1, 1, :_N, 0]

    dis_vv = jnp.where(deg_vv > 0, deg_vv ** -0.5, 0.0)
    x16 = jnp.concatenate([game_x, jnp.ones((_N, 1), f32),
                           jnp.zeros((_N, 10), f32)], axis=1)
    xp16 = dis_vv[:, None] * x16
    tr_e = _N + (jnp.arange(src_vv.shape[0], dtype=jnp.int32) % _TRASH)
    drel = [_pad_dst(jnp.where(et == r, dst_vv.astype(jnp.int32), tr_e))
            for r in range(_NREL)]
    gameo = _segq_call(jnp.concatenate([x16, x16, x16, xp16], axis=0),
                       srcg, _dst4(drel[0], drel[1], drel[2], dhop), ea16)
    rgcn = game_x @ p['rgcn_root'] + p['rgcn_b']
    quads = [gameo[0, 0], gameo[0, 1], gameo[1, 0], gameo[1, 1]]
    for r in range(_NREL):
        S_r = quads[r][:_N, :5]
        cnt_r = quads[r][:_N, 5]
        rgcn = rgcn + (S_r / jnp.clip(cnt_r, 1.0)[:, None]) @ p['rgcn_W'][r]

    zeros3n = jnp.zeros((3 * _N, 16), f32)
    tr_full = _pad_dst(tr_e)
    dhop4 = _dst4(dhop, tr_full, tr_full, tr_full)
    tag = game_x @ p['tag1_W'][0]
    h = dis_vv[:, None] * quads[3][:_N, :5]
    tag = tag + h @ p['tag1_W'][1]
    for k in range(2, _K + 1):
        hp16 = jnp.concatenate([dis_vv[:, None] * h,
                                jnp.zeros((_N, 11), f32)], axis=1)
        hsum = _segq_call(jnp.concatenate([hp16, zeros3n], axis=0),
                          srcg, dhop4, ea16)
        h = dis_vv[:, None] * hsum[0, 0, :_N, :5]
        tag = tag + h @ p['tag1_W'][k]
    gx = rgcn + tag + p['tag1_b']

    t4_gx = _quad_table(gx)
    agg_w = _quad_out(_segqw_call(t4_gx, srch, dhs4, ea16))
    agg_u = _quad_out(_segq_call(t4_gx, srch, dhs4, ea16))

    sx = jax.nn.relu(agg_w @ p['gc3_Wrel'] + p['gc3_brel']
                     + state_x @ p['gc3_Wroot'])
    mean_hs = agg_u / jnp.clip(cnt_hs, 1.0)[:, None]
    sx = jax.nn.relu(mean_hs @ p['s32_Wl'] + p['s32_bl'] + sx @ p['s32_Wr'])

    mean_in = _seg64(gx, srci, din4, ea16) / jnp.clip(cnt_in, 1.0)[:, None]
    sx = jax.nn.relu(mean_in @ p['s4_Wl'] + p['s4_bl'] + sx @ p['s4_Wr'])
    sx = jax.nn.relu(mean_in @ p['s42_Wl'] + p['s42_bl'] + sx @ p['s42_Wr'])

    dis_ss = jnp.where(cnt_ss > 0, cnt_ss ** -0.5, 0.0)
    tag2 = sx @ p['tag2_W'][0]
    h = sx
    for k in range(1, _K + 1):
        hp = dis_ss[:, None] * h
        h = dis_ss[:, None] * _seg64(hp, srcs, dss4, ea16)
        tag2 = tag2 + h @ p['tag2_W'][k]
    sx = jax.nn.relu(tag2 + p['tag2_b'])

    mean5 = _seg64(sx, srcs, dss4, ea16) / jnp.clip(cnt_ss, 1.0)[:, None]
    sx = jax.nn.relu(mean5 @ p['s5_Wl'] + p['s5_bl'] + sx @ p['s5_Wr'])

    return _final_linear_pallas(sx, p['lin_W'], p['lin_b'])


# speedup vs baseline: 17.0527x; 1.1223x over previous
import functools

import jax
import jax.numpy as jnp
from jax import lax
from jax.experimental import pallas as pl
from jax.experimental.pallas import tpu as pltpu
from jax.experimental.pallas import tpu_sc as plsc

_N = 50000
_NREL = 3
_K = 3
_NACC = 50176
_TRASH = _NACC - _N

_MESH = plsc.VectorSubcoreMesh(core_axis_name="c", subcore_axis_name="s")
_PARAMS = pltpu.CompilerParams(use_tc_tiling_on_sc=False)

_EP = 819200
_ROWS = _EP // 128


def _zeros16():
    return jnp.zeros((16,), jnp.float32)


_WR = 8


def _segf_body(t4_ref, src_ref, dst_ref, ea_ref, flag_ref, out_ref,
               src_a, src_b, dst_a, dst_b, rows_a, rows_b, w16_a, w16_b,
               fl_s, zb_v, acc_sh, semg_a, semg_b, sems_a, sems_b):
    c = lax.axis_index("c")
    s = lax.axis_index("s")
    srcs = [src_a, src_b]
    dsts = [dst_a, dst_b]
    rows = [rows_a, rows_b]
    w16s = [w16_a, w16_b]
    semg = [semg_a, semg_b]
    sems = [sems_a, sems_b]
    for i in range(64):
        zb_v[i, pl.ds(0, 16)] = _zeros16()
    pltpu.sync_copy(flag_ref, fl_s)
    wflag = fl_s[pl.ds(0, 16)][0] != 0
    tb = s * (_NACC // 16)
    row0 = s * (_ROWS // 16)
    nw = _ROWS // 16 // _WR

    def load_fire(w, b):
        r = row0 + w * _WR
        pltpu.sync_copy(src_ref.at[pl.ds(r, _WR)], srcs[b])

        @pl.when(wflag)
        def _():
            pltpu.sync_copy(ea_ref.at[pl.ds(r * 128, _WR * 128)], w16s[b])
        for i in range(_WR):
            for g in range(8):
                srcs[b][i, pl.ds(g * 16, 16)] = (
                    srcs[b][i, pl.ds(g * 16, 16)] + load_fire.off)
        for j in range(_WR):
            pltpu.async_copy(t4_ref.at[srcs[b].at[j]],
                             rows[b].at[pl.ds(j * 128, 128)], semg[b])

    def load_dst(w, b, rnd):
        r = row0 + w * _WR
        pltpu.sync_copy(dst_ref.at[rnd, c, pl.ds(r, _WR)], dsts[b])

    def finish(b):
        for j in range(_WR):
            pltpu.make_async_copy(t4_ref.at[srcs[b].at[j]],
                                  rows[b].at[pl.ds(j * 128, 128)],
                                  semg[b]).wait()

        @pl.when(wflag)
        def _():
            def ml(e, _):
                rows[b][e, :] = rows[b][e, :] * w16s[b][e, :]
                return 0
            lax.fori_loop(0, _WR * 128, ml, 0, unroll=8)
        for j in range(_WR):
            pltpu.async_copy(rows[b].at[pl.ds(j * 128, 128)],
                             acc_sh.at[dsts[b].at[j]], sems[b], add=True)

    def drain_sc(b):
        for j in range(_WR):
            pltpu.make_async_copy(rows[b].at[pl.ds(j * 128, 128)],
                                  acc_sh.at[dsts[b].at[j]], sems[b]).wait()

    for rnd in range(2):
        def zl(j, _):
            pltpu.sync_copy(zb_v, acc_sh.at[pl.ds(tb + j * 64, 64)])
            return 0
        lax.fori_loop(0, _NACC // 16 // 64, zl, 0)
        plsc.subcore_barrier()
        load_fire.off = (c + 2 * rnd) * _N

        load_dst(0, 0, rnd)
        load_fire(0, 0)

        def body(t, _):
            @pl.when(t > 0)
            def _():
                drain_sc(1)
            load_dst(2 * t + 1, 1, rnd)
            load_fire(2 * t + 1, 1)
            finish(0)

            @pl.when(t < nw // 2 - 1)
            def _():
                drain_sc(0)
                load_dst(2 * t + 2, 0, rnd)
                load_fire(2 * t + 2, 0)
            finish(1)
            return 0
        lax.fori_loop(0, nw // 2, body, 0)
        drain_sc(0)
        drain_sc(1)
        plsc.subcore_barrier()
        pltpu.sync_copy(acc_sh.at[pl.ds(tb, _NACC // 16)],
                        out_ref.at[rnd, c, pl.ds(tb, _NACC // 16)])
        plsc.subcore_barrier()


_segf = functools.partial(
    pl.kernel,
    out_type=jax.ShapeDtypeStruct((2, 2, _NACC, 16), jnp.float32),
    mesh=_MESH, compiler_params=_PARAMS,
    scratch_types=[
        pltpu.VMEM((_WR, 128), jnp.int32),
        pltpu.VMEM((_WR, 128), jnp.int32),
        pltpu.VMEM((_WR, 128), jnp.int32),
        pltpu.VMEM((_WR, 128), jnp.int32),
        pltpu.VMEM((_WR * 128, 16), jnp.float32),
        pltpu.VMEM((_WR * 128, 16), jnp.float32),
        pltpu.VMEM((_WR * 128, 16), jnp.float32),
        pltpu.VMEM((_WR * 128, 16), jnp.float32),
        pltpu.VMEM((16,), jnp.int32),
        pltpu.VMEM((64, 16), jnp.float32),
        pltpu.VMEM_SHARED((_NACC, 16), jnp.float32),
        pltpu.SemaphoreType.DMA,
        pltpu.SemaphoreType.DMA,
        pltpu.SemaphoreType.DMA,
        pltpu.SemaphoreType.DMA,
    ],
)(_segf_body)

def _segq_call(t4, src_p, dst4, ea):
    return _segf(t4, src_p, dst4, ea, jnp.zeros((16,), jnp.int32))


def _segqw_call(t4, src_p, dst4, ea16):
    return _segf(t4, src_p, dst4, ea16, jnp.ones((16,), jnp.int32))


def _final_linear_pallas(sx, W, b):
    n, hid = sx.shape
    out_dim = W.shape[1]
    blk = 1000

    def body(x_ref, w_ref, b_ref, o_ref):
        o_ref[...] = x_ref[...] @ w_ref[...] + b_ref[...]

    return pl.pallas_call(
        body,
        grid=(n // blk,),
        in_specs=[
            pl.BlockSpec((blk, hid), lambda i: (i, 0)),
            pl.BlockSpec((hid, out_dim), lambda i: (0, 0)),
            pl.BlockSpec((1, out_dim), lambda i: (0, 0)),
        ],
        out_specs=pl.BlockSpec((blk, out_dim), lambda i: (i, 0)),
        out_shape=jax.ShapeDtypeStruct((n, out_dim), jnp.float32),
    )(sx, W, b.reshape(1, -1))


def _pad_src(src):
    pad = _EP - src.shape[0]
    ps = jnp.arange(pad, dtype=jnp.int32) % _N
    return jnp.concatenate([src.astype(jnp.int32), ps]).reshape(_ROWS, 128)


def _pad_dst(dst):
    pad = _EP - dst.shape[0]
    pd = _N + jnp.arange(pad, dtype=jnp.int32) % _TRASH
    return jnp.concatenate([dst.astype(jnp.int32), pd]).reshape(_ROWS, 128)


def _dst4(d00, d01, d10, d11):
    return jnp.stack([jnp.stack([d00, d01]), jnp.stack([d10, d11])])


def _quad_table(x):
    return jnp.concatenate([x[:, 0:16], x[:, 16:32], x[:, 32:48],
                            x[:, 48:64]], axis=0)


def _quad_out(o):
    return jnp.concatenate([o[0, 0, :_N], o[0, 1, :_N],
                            o[1, 0, :_N], o[1, 1, :_N]], axis=1)


def _seg64(x, src_p, dst4, ea):
    return _quad_out(_segq_call(_quad_table(x), src_p, dst4, ea))


def kernel(game_x, state_x, edge_index_v_v, edge_type_v_v,
           edge_index_history_v_s, edge_attr_history_v_s,
           edge_index_in_v_s, edge_index_s_s, params):
    p = params
    f32 = jnp.float32
    src_vv, dst_vv = edge_index_v_v[0], edge_index_v_v[1]
    src_hs, dst_hs = edge_index_history_v_s[0], edge_index_history_v_s[1]
    src_in, dst_in = edge_index_in_v_s[0], edge_index_in_v_s[1]
    src_ss, dst_ss = edge_index_s_s[0], edge_index_s_s[1]
    et = edge_type_v_v.astype(jnp.int32)

    srcg = _pad_src(src_vv)
    srch = _pad_src(src_hs)
    srci = _pad_src(src_in)
    srcs = _pad_src(src_ss)
    dhop = _pad_dst(dst_vv)
    dhs = _pad_dst(dst_hs)
    din = _pad_dst(dst_in)
    dss = _pad_dst(dst_ss)
    dhs4 = _dst4(dhs, dhs, dhs, dhs)
    din4 = _dst4(din, din, din, din)
    dss4 = _dst4(dss, dss, dss, dss)
    padf = _EP - src_hs.shape[0]
    ea16 = jnp.broadcast_to(
        jnp.concatenate([edge_attr_history_v_s.astype(f32),
                         jnp.zeros((padf,), f32)])[:, None],
        (_EP, 16))

    ones4 = jnp.ones((4 * _N, 16), f32)
    cnto = _segq_call(ones4, srch, _dst4(dhs, din, dss, dhop), ea16)
    cnt_hs = cnto[0, 0, :_N, 0]
    cnt_in = cnto[0, 1, :_N, 0]
    cnt_ss = cnto[1, 0, :_N, 0]
    deg_vv = cnto[1, 1, :_N, 0]

    dis_vv = jnp.where(deg_vv > 0, deg_vv ** -0.5, 0.0)
    x16 = jnp.concatenate([game_x, jnp.ones((_N, 1), f32),
                           jnp.zeros((_N, 10), f32)], axis=1)
    xp16 = dis_vv[:, None] * x16
    tr_e = _N + (jnp.arange(src_vv.shape[0], dtype=jnp.int32) % _TRASH)
    drel = [_pad_dst(jnp.where(et == r, dst_vv.astype(jnp.int32), tr_e))
            for r in range(_NREL)]
    gameo = _segq_call(jnp.concatenate([x16, x16, x16, xp16], axis=0),
                       srcg, _dst4(drel[0], drel[1], drel[2], dhop), ea16)
    rgcn = game_x @ p['rgcn_root'] + p['rgcn_b']
    quads = [gameo[0, 0], gameo[0, 1], gameo[1, 0], gameo[1, 1]]
    for r in range(_NREL):
        S_r = quads[r][:_N, :5]
        cnt_r = quads[r][:_N, 5]
        rgcn = rgcn + (S_r / jnp.clip(cnt_r, 1.0)[:, None]) @ p['rgcn_W'][r]

    zeros3n = jnp.zeros((3 * _N, 16), f32)
    tr_full = _pad_dst(tr_e)
    dhop4 = _dst4(dhop, tr_full, tr_full, tr_full)
    tag = game_x @ p['tag1_W'][0]
    h = dis_vv[:, None] * quads[3][:_N, :5]
    tag = tag + h @ p['tag1_W'][1]
    for k in range(2, _K + 1):
        hp16 = jnp.concatenate([dis_vv[:, None] * h,
                                jnp.zeros((_N, 11), f32)], axis=1)
        hsum = _segq_call(jnp.concatenate([hp16, zeros3n], axis=0),
                          srcg, dhop4, ea16)
        h = dis_vv[:, None] * hsum[0, 0, :_N, :5]
        tag = tag + h @ p['tag1_W'][k]
    gx = rgcn + tag + p['tag1_b']

    t4_gx = _quad_table(gx)
    agg_w = _quad_out(_segqw_call(t4_gx, srch, dhs4, ea16))
    agg_u = _quad_out(_segq_call(t4_gx, srch, dhs4, ea16))

    sx = jax.nn.relu(agg_w @ p['gc3_Wrel'] + p['gc3_brel']
                     + state_x @ p['gc3_Wroot'])
    mean_hs = agg_u / jnp.clip(cnt_hs, 1.0)[:, None]
    sx = jax.nn.relu(mean_hs @ p['s32_Wl'] + p['s32_bl'] + sx @ p['s32_Wr'])

    mean_in = _seg64(gx, srci, din4, ea16) / jnp.clip(cnt_in, 1.0)[:, None]
    sx = jax.nn.relu(mean_in @ p['s4_Wl'] + p['s4_bl'] + sx @ p['s4_Wr'])
    sx = jax.nn.relu(mean_in @ p['s42_Wl'] + p['s42_bl'] + sx @ p['s42_Wr'])

    dis_ss = jnp.where(cnt_ss > 0, cnt_ss ** -0.5, 0.0)
    tag2 = sx @ p['tag2_W'][0]
    h = sx
    for k in range(1, _K + 1):
        hp = dis_ss[:, None] * h
        h = dis_ss[:, None] * _seg64(hp, srcs, dss4, ea16)
        tag2 = tag2 + h @ p['tag2_W'][k]
    sx = jax.nn.relu(tag2 + p['tag2_b'])

    mean5 = _seg64(sx, srcs, dss4, ea16) / jnp.clip(cnt_ss, 1.0)[:, None]
    sx = jax.nn.relu(mean5 @ p['s5_Wl'] + p['s5_bl'] + sx @ p['s5_Wr'])

    return _final_linear_pallas(sx, p['lin_W'], p['lin_b'])
